# baseline (speedup 1.0000x reference)
import functools

import jax
import jax.numpy as jnp
from jax import lax
from jax.experimental import pallas as pl
from jax.experimental.pallas import tpu as pltpu
from jax.experimental.pallas import tpu_sc as plsc

_EPS = 1e-10
_KEY_SEED = 42


def _rotl(x, r):
    return lax.shift_left(x, jnp.uint32(r)) | lax.shift_right_logical(
        x, jnp.uint32(32 - r))


def _threefry_bits(shape, n_total_cols):
    k0 = jnp.uint32(0)
    k1 = jnp.uint32(_KEY_SEED)
    ks2 = jnp.uint32(0x1BD11BDA) ^ k0 ^ k1
    ks = (k0, k1, ks2)
    lin = (lax.broadcasted_iota(jnp.uint32, shape, 1)
           + lax.broadcasted_iota(jnp.uint32, shape, 0)
           * jnp.uint32(n_total_cols))
    x0 = jnp.full(shape, k0, jnp.uint32)
    x1 = lin + ks[1]
    rot_a = (13, 15, 26, 6)
    rot_b = (17, 29, 16, 24)
    for i in range(5):
        for r in (rot_a if i % 2 == 0 else rot_b):
            x0 = x0 + x1
            x1 = _rotl(x1, r) ^ x0
        x0 = x0 + ks[(i + 1) % 3]
        x1 = x1 + ks[(i + 2) % 3] + jnp.uint32(i + 1)
    return x0 ^ x1


def _uniform_from_bits(bits):
    f = lax.bitcast_convert_type(
        lax.shift_right_logical(bits, jnp.uint32(9)) | jnp.uint32(0x3F800000),
        jnp.float32) - 1.0
    minval = jnp.float32(_EPS)
    maxval = jnp.float32(1.0 - _EPS)
    return jnp.maximum(minval, f * (maxval - minval) + minval)


def _mlp_body(emb_ref, w1_ref, b1_ref, w2_ref, b2_ref, w3_ref, b3_ref,
              out_ref):
    hT = lax.dot_general(w1_ref[...], emb_ref[...], (((1,), (1,)), ((), ())),
                         preferred_element_type=jnp.float32)
    hT = jnp.maximum(hT + b1_ref[...], 0.0)
    h2T = lax.dot_general(w2_ref[...], hT, (((1,), (0,)), ((), ())),
                          preferred_element_type=jnp.float32)
    h2T = jnp.maximum(h2T + b2_ref[...], 0.0)
    cf = lax.dot_general(w3_ref[...], h2T, (((1,), (0,)), ((), ())),
                         preferred_element_type=jnp.float32) + b3_ref[0, 0]
    n = cf.shape[1]
    u = _uniform_from_bits(_threefry_bits((2, n), n))
    u0 = u[0:1, :]
    u1 = u[1:2, :]
    noise = -jnp.log(jnp.log(u1) / jnp.log(u0) + _EPS)
    logits = cf + noise
    probs = 1.0 / (1.0 + jnp.exp(-logits))
    out_ref[pl.ds(0, n)] = probs[0]
    out_ref[pl.ds(n, out_ref.shape[0] - n)] = jnp.zeros(
        (out_ref.shape[0] - n,), jnp.float32)


def _cluster_probs(emb, W1, b1, W2, b2, W3, b3, n_pad):
    n, d = emb.shape
    return pl.pallas_call(
        _mlp_body,
        out_shape=jax.ShapeDtypeStruct((n_pad,), jnp.float32),
    )(emb, W1, b1.reshape(d, 1), W2, b2.reshape(d, 1),
      W3, b3.reshape(1, 1))


@functools.cache
def _make_gather(batch: int, table_pad: int):
    info = plsc.get_sparse_core_info()
    nc, ns = info.num_cores, info.num_subcores
    nw = nc * ns
    bpw = batch // nw
    mesh = plsc.VectorSubcoreMesh(core_axis_name="c", subcore_axis_name="s")

    chunk = 128
    nchunks = bpw // chunk

    @functools.partial(
        pl.kernel,
        mesh=mesh,
        out_type=jax.ShapeDtypeStruct((batch,), jnp.float32),
        scratch_types=[
            pltpu.VMEM_SHARED((table_pad,), jnp.float32),
            pltpu.VMEM((bpw,), jnp.int32),
            pltpu.VMEM((bpw,), jnp.float32),
            pltpu.SemaphoreType.DMA,
            pltpu.SemaphoreType.DMA,
        ],
    )
    def gather_k(table_hbm, idx_hbm, out_hbm, table_sh, idx_v, out_v, sem,
                 idx_sem):
        wid = lax.axis_index("s") * nc + lax.axis_index("c")
        base = wid * bpw
        idx_cp = pltpu.async_copy(idx_hbm.at[pl.ds(base, bpw)], idx_v,
                                  idx_sem)

        @pl.when(lax.axis_index("s") == 0)
        def _():
            pltpu.sync_copy(table_hbm, table_sh)

        plsc.subcore_barrier()
        idx_cp.wait()
        descs = []
        for j in range(nchunks):
            sl = pl.ds(j * chunk, chunk)
            descs.append(
                pltpu.async_copy(table_sh.at[idx_v.at[sl]], out_v.at[sl],
                                 sem))
        for d in descs:
            d.wait()
        pltpu.sync_copy(out_v, out_hbm.at[pl.ds(base, bpw)])

    return gather_k


def kernel(feats, cluster_ids, emb, W1, b1, W2, b2, W3, b3):
    n_clusters = emb.shape[0]
    batch = cluster_ids.shape[0]
    table_pad = ((n_clusters + 1023) // 1024) * 1024
    table = _cluster_probs(emb, W1, b1, W2, b2, W3, b3, table_pad)
    out = _make_gather(batch, table_pad)(table, cluster_ids[:, 0])
    return out[:, None]


# speedup vs baseline: 1.1374x; 1.1374x over previous
import functools

import jax
import jax.numpy as jnp
from jax import lax
from jax.experimental import pallas as pl
from jax.experimental.pallas import tpu as pltpu
from jax.experimental.pallas import tpu_sc as plsc

_EPS = 1e-10
_KEY_SEED = 42


def _rotl(x, r):
    return lax.shift_left(x, jnp.uint32(r)) | lax.shift_right_logical(
        x, jnp.uint32(32 - r))


def _threefry_bits(shape, n_total_cols):
    k0 = jnp.uint32(0)
    k1 = jnp.uint32(_KEY_SEED)
    ks2 = jnp.uint32(0x1BD11BDA) ^ k0 ^ k1
    ks = (k0, k1, ks2)
    lin = (lax.broadcasted_iota(jnp.uint32, shape, 1)
           + lax.broadcasted_iota(jnp.uint32, shape, 0)
           * jnp.uint32(n_total_cols))
    x0 = jnp.full(shape, k0, jnp.uint32)
    x1 = lin + ks[1]
    rot_a = (13, 15, 26, 6)
    rot_b = (17, 29, 16, 24)
    for i in range(5):
        for r in (rot_a if i % 2 == 0 else rot_b):
            x0 = x0 + x1
            x1 = _rotl(x1, r) ^ x0
        x0 = x0 + ks[(i + 1) % 3]
        x1 = x1 + ks[(i + 2) % 3] + jnp.uint32(i + 1)
    return x0 ^ x1


def _uniform_from_bits(bits):
    f = lax.bitcast_convert_type(
        lax.shift_right_logical(bits, jnp.uint32(9)) | jnp.uint32(0x3F800000),
        jnp.float32) - 1.0
    minval = jnp.float32(_EPS)
    maxval = jnp.float32(1.0 - _EPS)
    return jnp.maximum(minval, f * (maxval - minval) + minval)


def _mlp_body(emb_ref, w1_ref, b1_ref, w2_ref, b2_ref, w3_ref, b3_ref,
              out_ref):
    hT = lax.dot_general(w1_ref[...], emb_ref[...], (((1,), (1,)), ((), ())),
                         preferred_element_type=jnp.float32)
    hT = jnp.maximum(hT + b1_ref[...].T, 0.0)
    h2T = lax.dot_general(w2_ref[...], hT, (((1,), (0,)), ((), ())),
                          preferred_element_type=jnp.float32)
    h2T = jnp.maximum(h2T + b2_ref[...].T, 0.0)
    cf = lax.dot_general(w3_ref[...], h2T, (((1,), (0,)), ((), ())),
                         preferred_element_type=jnp.float32) + b3_ref[0, 0]
    n = cf.shape[1]
    u = _uniform_from_bits(_threefry_bits((2, n), n))
    u0 = u[0:1, :]
    u1 = u[1:2, :]
    noise = -jnp.log(jnp.log(u1) / jnp.log(u0) + _EPS)
    logits = cf + noise
    probs = 1.0 / (1.0 + jnp.exp(-logits))
    out_ref[pl.ds(0, n)] = probs[0]
    out_ref[pl.ds(n, out_ref.shape[0] - n)] = jnp.zeros(
        (out_ref.shape[0] - n,), jnp.float32)


def _cluster_probs(emb, W1, b1, W2, b2, W3, b3, n_pad):
    n, d = emb.shape
    return pl.pallas_call(
        _mlp_body,
        out_shape=jax.ShapeDtypeStruct((n_pad,), jnp.float32),
    )(emb, W1, b1.reshape(1, d), W2, b2.reshape(1, d),
      W3, b3.reshape(1, 1))


@functools.cache
def _make_gather(batch: int, table_pad: int):
    info = plsc.get_sparse_core_info()
    nc, ns = info.num_cores, info.num_subcores
    nw = nc * ns
    bpw = batch // nw
    mesh = plsc.VectorSubcoreMesh(core_axis_name="c", subcore_axis_name="s")

    chunk = 128
    nchunks = bpw // chunk

    @functools.partial(
        pl.kernel,
        mesh=mesh,
        out_type=jax.ShapeDtypeStruct((batch,), jnp.float32),
        scratch_types=[
            pltpu.VMEM_SHARED((table_pad,), jnp.float32),
            pltpu.VMEM((bpw,), jnp.int32),
            pltpu.VMEM((bpw,), jnp.float32),
            pltpu.SemaphoreType.DMA,
            pltpu.SemaphoreType.DMA,
        ],
    )
    def gather_k(table_hbm, idx_hbm, out_hbm, table_sh, idx_v, out_v, sem,
                 idx_sem):
        wid = lax.axis_index("s") * nc + lax.axis_index("c")
        base = wid * bpw
        idx_cp = pltpu.async_copy(idx_hbm.at[pl.ds(base, bpw)], idx_v,
                                  idx_sem)

        @pl.when(lax.axis_index("s") == 0)
        def _():
            pltpu.sync_copy(table_hbm, table_sh)

        plsc.subcore_barrier()
        idx_cp.wait()
        descs = []
        for j in range(nchunks):
            sl = pl.ds(j * chunk, chunk)
            descs.append(
                pltpu.async_copy(table_sh.at[idx_v.at[sl]], out_v.at[sl],
                                 sem))
        for d in descs:
            d.wait()
        pltpu.sync_copy(out_v, out_hbm.at[pl.ds(base, bpw)])

    return gather_k


def kernel(feats, cluster_ids, emb, W1, b1, W2, b2, W3, b3):
    n_clusters = emb.shape[0]
    batch = cluster_ids.shape[0]
    table_pad = ((n_clusters + 1023) // 1024) * 1024
    table = _cluster_probs(emb, W1, b1, W2, b2, W3, b3, table_pad)
    out = _make_gather(batch, table_pad)(table, cluster_ids[:, 0])
    return out[:, None]
